# baseline (speedup 1.0000x reference)
import functools

import jax
import jax.numpy as jnp
from jax import lax
from jax.experimental import pallas as pl
from jax.experimental.pallas import tpu as pltpu
from jax.experimental.pallas import tpu_sc as plsc

_P_THR = 0.55
_C_THR = 0.7
_W_FEAT = 1.0
_W_ATTN = 1.0

_T = 512
_D = 512
_NSEG = 256
_NQ = 4
_QT = _T // _NQ
_LSEG = 64
_TRASH = _LSEG
_BIG = 1 << 20
_CH = 32


def _seg_ids(a):
    pred = a > _P_THR
    pred_f = jnp.where(pred, 1.0, 0.0)
    r = lax.broadcasted_iota(jnp.int32, (_T, _T), 0)
    c = lax.broadcasted_iota(jnp.int32, (_T, _T), 1)
    shift = jnp.where(r + 1 == c, 1.0, 0.0)
    triu = jnp.where(r <= c, 1.0, 0.0)
    prev_f = jnp.dot(pred_f, shift, preferred_element_type=jnp.float32)
    start_f = pred_f * (1.0 - prev_f)
    cum = jnp.dot(start_f, triu, preferred_element_type=jnp.float32)
    seg = cum.astype(jnp.int32) - 1
    col = lax.broadcasted_iota(jnp.int32, (1, _T), 1)
    return pred, pred_f, seg, col


def _quarter_firsts(pred, seg, col):
    segm = jnp.where(pred, seg, _BIG)
    gfs = []
    for q in range(_NQ):
        mask_q = (col >= q * _QT) & (col < (q + 1) * _QT)
        gfs.append(jnp.min(jnp.where(mask_q, segm, _BIG)))
    return gfs


def _prep_kernel(attn_ref, idx_ref):
    a = attn_ref[0]
    pred, _, seg, col = _seg_ids(a)
    rep = a > _C_THR
    gfs = _quarter_firsts(pred, seg, col)
    q_of_col = col // _QT
    gf_vec = jnp.full((1, _T), _BIG, jnp.int32)
    for q in range(_NQ):
        gf_vec = jnp.where(q_of_col == q, gfs[q], gf_vec)
    l = seg - gf_vec
    idx_p = jnp.where(pred, l, _TRASH)
    idx_r = jnp.where(rep, l, _TRASH)
    idx_ref[0] = jnp.concatenate([idx_p, idx_r], axis=0)


def _sc_body(feat_hbm, idx_hbm, out_hbm, acc, idxb, stage):
    c = lax.axis_index("c")
    s = lax.axis_index("s")
    w = c * 16 + s
    b = w // _NQ
    q = w % _NQ

    pltpu.sync_copy(idx_hbm.at[w], idxb)

    zero16 = jnp.zeros((16,), jnp.float32)
    nrows = 2 * (_LSEG + 1)

    def zrow(i, carry):
        for j in range(_D // 16):
            acc[pl.ds(i * _D + j * 16, 16)] = zero16
        return carry

    lax.fori_loop(0, nrows, zrow, 0)

    rep_off = (_LSEG + 1) * _D

    def chunk_body(k, carry):
        pltpu.sync_copy(
            feat_hbm.at[b, pl.ds(q * _QT * _D + k * (_CH * _D), _CH * _D)], stage
        )
        for g in range(_CH // 16):
            ipv = idxb[pl.ds(k * _CH + g * 16, 16)]
            irv = idxb[pl.ds(_QT + k * _CH + g * 16, 16)]
            for r2 in range(16):
                r = g * 16 + r2
                po = ipv[r2] * _D
                ro = rep_off + irv[r2] * _D
                for j in range(_D // 16):
                    v = stage[pl.ds(r * _D + j * 16, 16)]
                    plsc.addupdate(acc.at[pl.ds(po + j * 16, 16)], v)
                    plsc.addupdate(acc.at[pl.ds(ro + j * 16, 16)], v)
        return carry

    lax.fori_loop(0, _QT // _CH, chunk_body, 0)

    half = _LSEG * _D
    pltpu.sync_copy(acc.at[pl.ds(0, half)], out_hbm.at[w, pl.ds(0, half)])
    pltpu.sync_copy(acc.at[pl.ds(rep_off, half)], out_hbm.at[w, pl.ds(half, half)])


def _sc_scatter(feat_flat, idx_w):
    run = pl.kernel(
        _sc_body,
        out_type=jax.ShapeDtypeStruct((32, 2 * _LSEG * _D), jnp.float32),
        mesh=plsc.VectorSubcoreMesh(
            core_axis_name="c", subcore_axis_name="s", num_cores=2, num_subcores=16
        ),
        scratch_types=[
            pltpu.VMEM((2 * (_LSEG + 1) * _D,), jnp.float32),
            pltpu.VMEM((2 * _QT,), jnp.int32),
            pltpu.VMEM((_CH * _D,), jnp.float32),
        ],
    )
    return run(feat_flat, idx_w)


def _finalize_kernel(attn_ref, acc_ref, out_ref, sacc_ref):
    bidx = pl.program_id(0)
    nb = pl.num_programs(0)

    @pl.when(bidx == 0)
    def _init():
        sacc_ref[0] = 0.0
        sacc_ref[1] = 0.0
        sacc_ref[2] = 0.0

    a = attn_ref[0]
    pred, pred_f, seg, col = _seg_ids(a)
    rep_f = jnp.where(a > _C_THR, 1.0, 0.0)
    gfs = _quarter_firsts(pred, seg, col)

    row_ids = lax.broadcasted_iota(jnp.int32, (_NSEG, _T), 0)
    m = jnp.where((row_ids == seg) & pred, 1.0, 0.0)
    m_rep = m * rep_f

    counts = jnp.sum(m, axis=1, keepdims=True)
    rep_counts = jnp.sum(m_rep, axis=1, keepdims=True)
    sum_a = jnp.sum(m * a, axis=1, keepdims=True)
    sum_a2 = jnp.sum(m * (a * a), axis=1, keepdims=True)

    valid = counts > 0.0
    counts_safe = jnp.where(valid, counts, 1.0)
    mean_a = sum_a / counts_safe
    var = sum_a2 / counts_safe - mean_a * mean_a
    nprop = jnp.sum(jnp.where(valid, 1.0, 0.0))
    video_loss = jnp.sum(jnp.where(valid, var, 0.0))
    attn_contrib = jnp.where(nprop > 0.0, video_loss / jnp.maximum(nprop, 1.0), 0.0)

    r256 = lax.broadcasted_iota(jnp.int32, (_NSEG, _LSEG), 0)
    c64 = lax.broadcasted_iota(jnp.int32, (_NSEG, _LSEG), 1)
    acc_p = jnp.zeros((_NSEG, _D), jnp.float32)
    acc_r = jnp.zeros((_NSEG, _D), jnp.float32)
    for q in range(_NQ):
        mq = jnp.where(r256 == gfs[q] + c64, 1.0, 0.0)
        acc_p = acc_p + jnp.dot(mq, acc_ref[0, q, 0], preferred_element_type=jnp.float32)
        acc_r = acc_r + jnp.dot(mq, acc_ref[0, q, 1], preferred_element_type=jnp.float32)

    has_rep = valid & (rep_counts > 0.0)
    rep_safe = jnp.where(has_rep, rep_counts, 1.0)
    diff = acc_p / counts_safe - acc_r / rep_safe
    mse = jnp.sum(diff * diff, axis=1, keepdims=True) / _D
    feat_contrib = jnp.sum(jnp.where(has_rep, mse, 0.0))
    cnt_contrib = jnp.sum(jnp.where(has_rep, 1.0, 0.0))

    sacc_ref[0] += feat_contrib
    sacc_ref[1] += cnt_contrib
    sacc_ref[2] += attn_contrib

    @pl.when(bidx == nb - 1)
    def _fin():
        fls = sacc_ref[0]
        fc = sacc_ref[1]
        feat_loss = jnp.where(fc > 0.0, fls / jnp.maximum(fc, 1.0), fls)
        out_ref[0, 0] = _W_FEAT * feat_loss + _W_ATTN * sacc_ref[2] / nb


def kernel(attn, feat):
    B = attn.shape[0]
    attn3 = attn.reshape(B, 1, _T)

    idx = pl.pallas_call(
        _prep_kernel,
        grid=(B,),
        in_specs=[pl.BlockSpec((1, 1, _T), lambda b: (b, 0, 0))],
        out_specs=pl.BlockSpec((1, 2, _T), lambda b: (b, 0, 0)),
        out_shape=jax.ShapeDtypeStruct((B, 2, _T), jnp.int32),
    )(attn3)

    idx_w = (
        idx.reshape(B, 2, _NQ, _QT)
        .transpose(0, 2, 1, 3)
        .reshape(B * _NQ, 2 * _QT)
    )
    feat_flat = feat.reshape(B, _T * _D)

    acc_w = _sc_scatter(feat_flat, idx_w)
    acc5 = acc_w.reshape(B, _NQ, 2, _LSEG, _D)

    out = pl.pallas_call(
        _finalize_kernel,
        grid=(B,),
        in_specs=[
            pl.BlockSpec((1, 1, _T), lambda b: (b, 0, 0)),
            pl.BlockSpec((1, _NQ, 2, _LSEG, _D), lambda b: (b, 0, 0, 0, 0)),
        ],
        out_specs=pl.BlockSpec(memory_space=pltpu.SMEM),
        out_shape=jax.ShapeDtypeStruct((1, 1), jnp.float32),
        scratch_shapes=[pltpu.SMEM((3,), jnp.float32)],
    )(attn3, acc5)
    return out[0, 0]
